# speedup vs baseline: 1.6240x; 1.6240x over previous
import functools

import jax
import jax.numpy as jnp
from jax import lax
from jax.experimental import pallas as pl
from jax.experimental.pallas import tpu as pltpu
from jax.experimental.pallas import tpu_sc as plsc

_NUM_TOKENS = 2048
_HIDDEN = 2048
_BATCH = 4

_info = plsc.get_sparse_core_info()
_NC = _info.num_cores
_NS = _info.num_subcores
_NW = _NC * _NS
_B = _BATCH * _NUM_TOKENS
_BPW = _B // _NW
_CH = 16
_NCH = _BPW // _CH

_mesh = plsc.VectorSubcoreMesh(core_axis_name="c", subcore_axis_name="s")


@functools.partial(
    pl.kernel,
    out_type=jax.ShapeDtypeStruct((_B, _HIDDEN), jnp.float32),
    mesh=_mesh,
    scratch_types=[
        pltpu.VMEM((_BPW,), jnp.int32),
        pltpu.VMEM((_CH, _HIDDEN), jnp.float32),
        pltpu.VMEM((_CH, _HIDDEN), jnp.float32),
        pltpu.SemaphoreType.DMA,
        pltpu.SemaphoreType.DMA,
        pltpu.SemaphoreType.DMA,
        pltpu.SemaphoreType.DMA,
    ],
)
def _gather_rows(idx_hbm, table_hbm, out_hbm, idx_v, buf0, buf1,
                 g0, g1, w0, w1):
    wid = lax.axis_index("s") * _NC + lax.axis_index("c")
    base = wid * _BPW
    pltpu.sync_copy(idx_hbm.at[pl.ds(base, _BPW)], idx_v)

    bufs = (buf0, buf1)
    gsems = (g0, g1)
    wsems = (w0, w1)

    def gather(c):
        return pltpu.async_copy(
            table_hbm.at[idx_v.at[pl.ds(c * _CH, _CH)]],
            bufs[c % 2], gsems[c % 2])

    def put(c):
        return pltpu.async_copy(
            bufs[c % 2], out_hbm.at[pl.ds(base + c * _CH, _CH)],
            wsems[c % 2])

    writes = [None] * _NCH
    pending = gather(0)
    for c in range(_NCH):
        if c + 1 < _NCH:
            if c >= 1:
                writes[c - 1].wait()
            nxt = gather(c + 1)
        pending.wait()
        writes[c] = put(c)
        if c + 1 < _NCH:
            pending = nxt
    writes[_NCH - 2].wait()
    writes[_NCH - 1].wait()


def kernel(prompts, prompt_weight):
    idx = prompts.reshape(-1).astype(jnp.int32)
    out = _gather_rows(idx, prompt_weight)
    return out.reshape(prompts.shape + (prompt_weight.shape[-1],))


# speedup vs baseline: 1.6567x; 1.0201x over previous
import functools

import jax
import jax.numpy as jnp
from jax import lax
from jax.experimental import pallas as pl
from jax.experimental.pallas import tpu as pltpu
from jax.experimental.pallas import tpu_sc as plsc

_NUM_TOKENS = 2048
_HIDDEN = 2048
_BATCH = 4

_info = plsc.get_sparse_core_info()
_NC = _info.num_cores
_NS = _info.num_subcores
_NW = _NC * _NS
_B = _BATCH * _NUM_TOKENS
_BPW = _B // _NW
_CH = 16
_NCH = _BPW // _CH
_NBUF = 3

_mesh = plsc.VectorSubcoreMesh(core_axis_name="c", subcore_axis_name="s")


@functools.partial(
    pl.kernel,
    out_type=jax.ShapeDtypeStruct((_B, _HIDDEN), jnp.float32),
    mesh=_mesh,
    scratch_types=(
        [pltpu.VMEM((_BPW,), jnp.int32)]
        + [pltpu.VMEM((_CH, _HIDDEN), jnp.float32)] * _NBUF
        + [pltpu.SemaphoreType.DMA] * (2 * _NBUF)
    ),
)
def _gather_rows(idx_hbm, table_hbm, out_hbm, idx_v, *scratch):
    bufs = scratch[:_NBUF]
    gsems = scratch[_NBUF:2 * _NBUF]
    wsems = scratch[2 * _NBUF:]
    wid = lax.axis_index("s") * _NC + lax.axis_index("c")
    base = wid * _BPW
    pltpu.sync_copy(idx_hbm.at[pl.ds(base, _BPW)], idx_v)

    def gather(c):
        return pltpu.async_copy(
            table_hbm.at[idx_v.at[pl.ds(c * _CH, _CH)]],
            bufs[c % _NBUF], gsems[c % _NBUF])

    def put(c):
        return pltpu.async_copy(
            bufs[c % _NBUF], out_hbm.at[pl.ds(base + c * _CH, _CH)],
            wsems[c % _NBUF])

    gets = [None] * _NCH
    writes = [None] * _NCH
    for c in range(_NBUF):
        gets[c] = gather(c)
    for c in range(_NCH):
        gets[c].wait()
        writes[c] = put(c)
        if c + _NBUF < _NCH:
            writes[c].wait()
            gets[c + _NBUF] = gather(c + _NBUF)
    for c in range(max(0, _NCH - _NBUF), _NCH):
        writes[c].wait()


def kernel(prompts, prompt_weight):
    idx = prompts.reshape(-1).astype(jnp.int32)
    out = _gather_rows(idx, prompt_weight)
    return out.reshape(prompts.shape + (prompt_weight.shape[-1],))
